# baseline (speedup 1.0000x reference)
import functools
import math

import jax
import jax.numpy as jnp
import numpy as np
from jax import lax
from jax.experimental import pallas as pl
from jax.experimental.pallas import tpu as pltpu
from jax.experimental.pallas import tpu_sc as plsc

N_LEVELS = 16
N_FEATURES = 2
MIN_RES = 16
MAX_RES = 2048
LOG2_T = 20
T = 1 << LOG2_T
HASH_MASK = T - 1
PRIME_I32 = np.int32(2654435761 - (1 << 32))
_GROWTH = math.exp((math.log(MAX_RES) - math.log(MIN_RES)) / (N_LEVELS - 1))
RES = [int(math.floor(MIN_RES * (_GROWTH ** l))) for l in range(N_LEVELS)]

D_ENC = N_LEVELS * N_FEATURES


def _sc_encode(uv_flat, tab):
    B = uv_flat.shape[0] // 2
    info = plsc.get_sparse_core_info()
    NC, NS, L = info.num_cores, info.num_subcores, info.num_lanes
    NW = NC * NS
    assert B % NW == 0
    PPW = B // NW
    C = 1024
    assert PPW % C == 0
    NCHUNK = PPW // C

    mesh = plsc.VectorSubcoreMesh(core_axis_name="c", subcore_axis_name="s")

    @functools.partial(
        pl.kernel,
        out_type=jax.ShapeDtypeStruct((D_ENC, B), jnp.float32),
        mesh=mesh,
        scratch_types=[
            pltpu.VMEM((2 * C,), jnp.float32),
            pltpu.VMEM((2, C), jnp.float32),
            pltpu.VMEM((2, C), jnp.float32),
            pltpu.VMEM((8 * C,), jnp.int32),
            pltpu.VMEM((8 * C,), jnp.int32),
            pltpu.VMEM((8 * C,), jnp.float32),
            pltpu.VMEM((8 * C,), jnp.float32),
            pltpu.VMEM((D_ENC, C), jnp.float32),
            pltpu.SemaphoreType.DMA,
            pltpu.SemaphoreType.DMA,
        ],
    )
    def enc_kernel(uv_hbm, tab_hbm, out_hbm, uv_v, wx_v, wy_v,
                   idx0_v, idx1_v, feat0_v, feat1_v, enc_v, sem0, sem1):
        wid = lax.axis_index("s") * NC + lax.axis_index("c")
        sems = (sem0, sem1)
        idxs = (idx0_v, idx1_v)
        feats = (feat0_v, feat1_v)

        def chunk_body(ci, _):
            base = wid * PPW + ci * C
            pltpu.sync_copy(uv_hbm.at[pl.ds(2 * base, 2 * C)], uv_v)

            def idx_pass(l, sl):
                res = float(RES[l])
                lofs2 = 2 * l * T

                def idx_body(i, _):
                    off = i * L
                    ou = (i // 8) * 256 + (i % 8) * L
                    uu = uv_v[pl.ds(ou, L)]
                    vv = uv_v[pl.ds(ou + 128, L)]
                    px = uu * res
                    py = vv * res
                    ix = px.astype(jnp.int32)
                    iy = py.astype(jnp.int32)
                    wx_v[sl, pl.ds(off, L)] = px - ix.astype(jnp.float32)
                    wy_v[sl, pl.ds(off, L)] = py - iy.astype(jnp.float32)
                    hy0 = iy * PRIME_I32
                    hy1 = hy0 + PRIME_I32
                    ix1 = ix + 1
                    r00 = (ix ^ hy0) & HASH_MASK
                    r10 = (ix1 ^ hy0) & HASH_MASK
                    r01 = (ix ^ hy1) & HASH_MASK
                    r11 = (ix1 ^ hy1) & HASH_MASK
                    e00 = r00 + (r00 & -128) + lofs2
                    e10 = r10 + (r10 & -128) + lofs2
                    e01 = r01 + (r01 & -128) + lofs2
                    e11 = r11 + (r11 & -128) + lofs2
                    iv = idxs[sl]
                    iv[pl.ds(0 * C + off, L)] = e00
                    iv[pl.ds(1 * C + off, L)] = e00 + 128
                    iv[pl.ds(2 * C + off, L)] = e10
                    iv[pl.ds(3 * C + off, L)] = e10 + 128
                    iv[pl.ds(4 * C + off, L)] = e01
                    iv[pl.ds(5 * C + off, L)] = e01 + 128
                    iv[pl.ds(6 * C + off, L)] = e11
                    iv[pl.ds(7 * C + off, L)] = e11 + 128
                    return 0

                lax.fori_loop(0, C // L, idx_body, 0)

            def fire(sl):
                return pltpu.async_copy(
                    tab_hbm.at[idxs[sl]], feats[sl], sems[sl])

            def blend_pass(l, sl):
                def blend_body(i, _):
                    off = i * L
                    wx = wx_v[sl, pl.ds(off, L)]
                    wy = wy_v[sl, pl.ds(off, L)]
                    for f in range(N_FEATURES):
                        fv = feats[sl]
                        f00 = fv[pl.ds((0 + f) * C + off, L)]
                        f10 = fv[pl.ds((2 + f) * C + off, L)]
                        f01 = fv[pl.ds((4 + f) * C + off, L)]
                        f11 = fv[pl.ds((6 + f) * C + off, L)]
                        a = f00 + wx * (f10 - f00)
                        b = f01 + wx * (f11 - f01)
                        enc_v[N_FEATURES * l + f, pl.ds(off, L)] = (
                            a + wy * (b - a))
                    return 0

                lax.fori_loop(0, C // L, blend_body, 0)

            idx_pass(0, 0)
            cps = [fire(0), None]
            for l in range(N_LEVELS):
                sl = l % 2
                sn = (l + 1) % 2
                if l + 1 < N_LEVELS:
                    idx_pass(l + 1, sn)
                cps[sl].wait()
                if l + 1 < N_LEVELS:
                    cps[sn] = fire(sn)
                blend_pass(l, sl)
            pltpu.sync_copy(enc_v, out_hbm.at[:, pl.ds(base, C)])
            return 0

        lax.fori_loop(0, NCHUNK, chunk_body, 0)

    return enc_kernel(uv_flat, tab)


def _mlp_call(enc_t, W0, W1, W2):
    B = enc_t.shape[1]
    BT = 4096
    assert B % BT == 0
    dn_t = (((0,), (0,)), ((), ()))

    def mlp_kernel(e_ref, w0_ref, w1_ref, w2_ref, o_ref):
        h = lax.dot_general(e_ref[...], w0_ref[...], dn_t,
                            preferred_element_type=jnp.float32)
        h = jnp.maximum(h, 0.0)
        h = jnp.dot(h, w1_ref[...], preferred_element_type=jnp.float32)
        h = jnp.maximum(h, 0.0)
        o = jnp.dot(h, w2_ref[...], preferred_element_type=jnp.float32)
        o_ref[...] = jax.nn.sigmoid(o)

    return pl.pallas_call(
        mlp_kernel,
        grid=(B // BT,),
        in_specs=[
            pl.BlockSpec((D_ENC, BT), lambda i: (0, i)),
            pl.BlockSpec((D_ENC, 64), lambda i: (0, 0)),
            pl.BlockSpec((64, 64), lambda i: (0, 0)),
            pl.BlockSpec((64, 3), lambda i: (0, 0)),
        ],
        out_specs=pl.BlockSpec((BT, 3), lambda i: (i, 0)),
        out_shape=jax.ShapeDtypeStruct((B, 3), jnp.float32),
    )(enc_t, W0, W1, W2)


def kernel(uv, tables, W0, W1, W2):
    B = uv.shape[0]
    uv_flat = (uv.reshape(B // 128, 128, 2)
               .transpose(0, 2, 1)
               .reshape(2 * B))
    tab = (tables.reshape(N_LEVELS, T // 128, 128, N_FEATURES)
           .transpose(0, 1, 3, 2)
           .reshape(N_LEVELS * T * N_FEATURES))
    enc_t = _sc_encode(uv_flat, tab)
    return _mlp_call(enc_t, W0, W1, W2)


# speedup vs baseline: 1.0725x; 1.0725x over previous
import functools
import math

import jax
import jax.numpy as jnp
import numpy as np
from jax import lax
from jax.experimental import pallas as pl
from jax.experimental.pallas import tpu as pltpu
from jax.experimental.pallas import tpu_sc as plsc

N_LEVELS = 16
N_FEATURES = 2
MIN_RES = 16
MAX_RES = 2048
LOG2_T = 20
T = 1 << LOG2_T
HASH_MASK = T - 1
PRIME_I32 = np.int32(2654435761 - (1 << 32))
_GROWTH = math.exp((math.log(MAX_RES) - math.log(MIN_RES)) / (N_LEVELS - 1))
RES = [int(math.floor(MIN_RES * (_GROWTH ** l))) for l in range(N_LEVELS)]

D_ENC = N_LEVELS * N_FEATURES


def _sc_encode(uv_flat, tab):
    B = uv_flat.shape[0] // 2
    info = plsc.get_sparse_core_info()
    NC, NS, L = info.num_cores, info.num_subcores, info.num_lanes
    NW = NC * NS
    assert B % NW == 0
    PPW = B // NW
    C = 512
    assert PPW % C == 0
    NCHUNK = PPW // C

    mesh = plsc.VectorSubcoreMesh(core_axis_name="c", subcore_axis_name="s")

    @functools.partial(
        pl.kernel,
        out_type=jax.ShapeDtypeStruct((D_ENC, B), jnp.float32),
        mesh=mesh,
        scratch_types=[
            pltpu.VMEM((2 * C,), jnp.float32),
            pltpu.VMEM((2, C), jnp.float32),
            pltpu.VMEM((2, C), jnp.float32),
            pltpu.VMEM((8 * C,), jnp.int32),
            pltpu.VMEM((8 * C,), jnp.int32),
            pltpu.VMEM((8 * C,), jnp.float32),
            pltpu.VMEM((8 * C,), jnp.float32),
            pltpu.VMEM((D_ENC, C), jnp.float32),
            pltpu.SemaphoreType.DMA,
            pltpu.SemaphoreType.DMA,
        ],
    )
    def enc_kernel(uv_hbm, tab_hbm, out_hbm, uv_v, wx_v, wy_v,
                   idx0_v, idx1_v, feat0_v, feat1_v, enc_v, sem0, sem1):
        wid = lax.axis_index("s") * NC + lax.axis_index("c")
        sems = (sem0, sem1)
        idxs = (idx0_v, idx1_v)
        feats = (feat0_v, feat1_v)

        def chunk_body(ci, _):
            base = wid * PPW + ci * C
            pltpu.sync_copy(uv_hbm.at[pl.ds(2 * base, 2 * C)], uv_v)

            def idx_pass(l, sl):
                res = float(RES[l])
                lofs2 = 2 * l * T

                def idx_body(i, _):
                    off = i * L
                    ou = (i // 8) * 256 + (i % 8) * L
                    uu = uv_v[pl.ds(ou, L)]
                    vv = uv_v[pl.ds(ou + 128, L)]
                    px = uu * res
                    py = vv * res
                    ix = px.astype(jnp.int32)
                    iy = py.astype(jnp.int32)
                    wx_v[sl, pl.ds(off, L)] = px - ix.astype(jnp.float32)
                    wy_v[sl, pl.ds(off, L)] = py - iy.astype(jnp.float32)
                    hy0 = iy * PRIME_I32
                    hy1 = hy0 + PRIME_I32
                    ix1 = ix + 1
                    r00 = (ix ^ hy0) & HASH_MASK
                    r10 = (ix1 ^ hy0) & HASH_MASK
                    r01 = (ix ^ hy1) & HASH_MASK
                    r11 = (ix1 ^ hy1) & HASH_MASK
                    e00 = r00 + (r00 & -128) + lofs2
                    e10 = r10 + (r10 & -128) + lofs2
                    e01 = r01 + (r01 & -128) + lofs2
                    e11 = r11 + (r11 & -128) + lofs2
                    iv = idxs[sl]
                    iv[pl.ds(0 * C + off, L)] = e00
                    iv[pl.ds(1 * C + off, L)] = e00 + 128
                    iv[pl.ds(2 * C + off, L)] = e10
                    iv[pl.ds(3 * C + off, L)] = e10 + 128
                    iv[pl.ds(4 * C + off, L)] = e01
                    iv[pl.ds(5 * C + off, L)] = e01 + 128
                    iv[pl.ds(6 * C + off, L)] = e11
                    iv[pl.ds(7 * C + off, L)] = e11 + 128
                    return 0

                lax.fori_loop(0, C // L, idx_body, 0)

            def fire(sl):
                return pltpu.async_copy(
                    tab_hbm.at[idxs[sl]], feats[sl], sems[sl])

            def blend_pass(l, sl):
                def blend_body(i, _):
                    off = i * L
                    wx = wx_v[sl, pl.ds(off, L)]
                    wy = wy_v[sl, pl.ds(off, L)]
                    for f in range(N_FEATURES):
                        fv = feats[sl]
                        f00 = fv[pl.ds((0 + f) * C + off, L)]
                        f10 = fv[pl.ds((2 + f) * C + off, L)]
                        f01 = fv[pl.ds((4 + f) * C + off, L)]
                        f11 = fv[pl.ds((6 + f) * C + off, L)]
                        a = f00 + wx * (f10 - f00)
                        b = f01 + wx * (f11 - f01)
                        enc_v[N_FEATURES * l + f, pl.ds(off, L)] = (
                            a + wy * (b - a))
                    return 0

                lax.fori_loop(0, C // L, blend_body, 0)

            idx_pass(0, 0)
            cps = [fire(0), None]
            for l in range(N_LEVELS):
                sl = l % 2
                sn = (l + 1) % 2
                if l + 1 < N_LEVELS:
                    idx_pass(l + 1, sn)
                cps[sl].wait()
                if l + 1 < N_LEVELS:
                    cps[sn] = fire(sn)
                blend_pass(l, sl)
            pltpu.sync_copy(enc_v, out_hbm.at[:, pl.ds(base, C)])
            return 0

        lax.fori_loop(0, NCHUNK, chunk_body, 0)

    return enc_kernel(uv_flat, tab)


def _mlp_call(enc_t, W0, W1, W2):
    B = enc_t.shape[1]
    BT = 4096
    assert B % BT == 0
    w0t = W0.T
    w1t = W1.T
    w2t = W2.T

    def mlp_kernel(e_ref, w0_ref, w1_ref, w2_ref, o_ref):
        h = jnp.dot(w0_ref[...], e_ref[...], preferred_element_type=jnp.float32)
        h = jnp.maximum(h, 0.0)
        h = jnp.dot(w1_ref[...], h, preferred_element_type=jnp.float32)
        h = jnp.maximum(h, 0.0)
        o = jax.nn.sigmoid(
            jnp.dot(w2_ref[...], h, preferred_element_type=jnp.float32))
        for k in range(BT // 128):
            for f in range(3):
                o_ref[pl.ds(k * 512 + f * 128, 128)] = o[f, k * 128:(k + 1) * 128]
            o_ref[pl.ds(k * 512 + 384, 128)] = jnp.zeros((128,), jnp.float32)

    return pl.pallas_call(
        mlp_kernel,
        grid=(B // BT,),
        in_specs=[
            pl.BlockSpec((D_ENC, BT), lambda i: (0, i)),
            pl.BlockSpec((64, D_ENC), lambda i: (0, 0)),
            pl.BlockSpec((64, 64), lambda i: (0, 0)),
            pl.BlockSpec((3, 64), lambda i: (0, 0)),
        ],
        out_specs=pl.BlockSpec((BT * 4,), lambda i: (i,)),
        out_shape=jax.ShapeDtypeStruct((B * 4,), jnp.float32),
    )(enc_t, w0t, w1t, w2t)


def kernel(uv, tables, W0, W1, W2):
    B = uv.shape[0]
    uv_flat = (uv.reshape(B // 128, 128, 2)
               .transpose(0, 2, 1)
               .reshape(2 * B))
    tab = (tables.reshape(N_LEVELS, T // 128, 128, N_FEATURES)
           .transpose(0, 1, 3, 2)
           .reshape(N_LEVELS * T * N_FEATURES))
    enc_t = _sc_encode(uv_flat, tab)
    out1d = _mlp_call(enc_t, W0, W1, W2)
    o4 = (out1d.reshape(B // 128, 4, 128)
          .transpose(0, 2, 1)
          .reshape(B, 4))
    return o4[:, :3]
